# baseline (speedup 1.0000x reference)
import functools

import jax
import jax.numpy as jnp
import numpy as np
from jax import lax
from jax.experimental import pallas as pl
from jax.experimental.pallas import tpu as pltpu
from jax.experimental.pallas import tpu_sc as plsc

_BN_EPS = 1e-5
_BIG = 1e30


def _fuse_layer(layer):
    s = layer['gamma'] / jnp.sqrt(1.0 + _BN_EPS)
    A = layer['W'].T * s[None, :]
    c = layer['b'] * s + layer['beta']
    return A, c


def _fuse_mlp(layers):
    out = []
    for l in layers:
        out.extend(_fuse_layer(l))
    return out


def _square_distance(src, dst):
    return (jnp.sum(src ** 2, -1)[:, :, None] + jnp.sum(dst ** 2, -1)[:, None, :]
            - 2.0 * jnp.einsum('bnc,bmc->bnm', src, dst))


def _index_points(points, idx):
    return jax.vmap(lambda p, i: p[i])(points, idx)


def _farthest_point_sample(xyz, npoint):
    B, N, _ = xyz.shape

    def body(i, carry):
        centroids, distance, farthest = carry
        centroids = centroids.at[:, i].set(farthest)
        centroid = jnp.take_along_axis(xyz, farthest[:, None, None], axis=1)
        dist = jnp.sum((xyz - centroid) ** 2, -1)
        distance = jnp.minimum(distance, dist)
        farthest = jnp.argmax(distance, -1).astype(jnp.int32)
        return centroids, distance, farthest

    centroids = jnp.zeros((B, npoint), jnp.int32)
    distance = jnp.full((B, N), 1e10, jnp.float32)
    farthest = jnp.zeros((B,), jnp.int32)
    centroids, _, _ = jax.lax.fori_loop(0, npoint, body,
                                        (centroids, distance, farthest))
    return centroids


def _query_ball_point(radius, nsample, xyz, new_xyz):
    B, N, _ = xyz.shape
    S = new_xyz.shape[1]
    sqrdists = _square_distance(new_xyz, xyz)
    group_idx = jnp.broadcast_to(
        jnp.arange(N, dtype=jnp.int32)[None, None, :], (B, S, N))
    group_idx = jnp.where(sqrdists > radius ** 2, N, group_idx)
    group_idx = jnp.sort(group_idx, axis=-1)[:, :, :nsample]
    group_first = jnp.broadcast_to(group_idx[:, :, :1], group_idx.shape)
    group_idx = jnp.where(group_idx == N, group_first, group_idx)
    return group_idx


def _apply_layer_xla(x, layer):
    h = jnp.einsum('...i,oi->...o', x, layer['W']) + layer['b']
    h = h * (layer['gamma'] / jnp.sqrt(1.0 + _BN_EPS)) + layer['beta']
    return jax.nn.relu(h)


def _sa_msg_xla(xyz, points, npoint, radius_list, nsample_list, branch_params,
                new_xyz=None):
    if new_xyz is None:
        fps_idx = _farthest_point_sample(xyz, npoint)
        new_xyz = _index_points(xyz, fps_idx)
    outs = []
    for radius, K, layers in zip(radius_list, nsample_list, branch_params):
        group_idx = _query_ball_point(radius, K, xyz, new_xyz)
        grouped_xyz = _index_points(xyz, group_idx) - new_xyz[:, :, None, :]
        if points is not None:
            h = jnp.concatenate([_index_points(points, group_idx), grouped_xyz], -1)
        else:
            h = grouped_xyz
        for layer in layers:
            h = _apply_layer_xla(h, layer)
        outs.append(jnp.max(h, axis=2))
    return new_xyz, jnp.concatenate(outs, -1)


def _fps_body(x_ref, y_ref, z_ref, cent_ref, nx_ref, ny_ref, nz_ref):
    B, N = x_ref.shape
    S = cent_ref.shape[1]
    x = x_ref[...]
    y = y_ref[...]
    z = z_ref[...]
    iota = jax.lax.broadcasted_iota(jnp.int32, (B, N), 1)
    iota_s = jax.lax.broadcasted_iota(jnp.int32, (B, S), 1)

    cent_ref[...] = jnp.zeros((B, S), jnp.float32)
    nx_ref[...] = jnp.zeros((B, S), jnp.float32)
    ny_ref[...] = jnp.zeros((B, S), jnp.float32)
    nz_ref[...] = jnp.zeros((B, S), jnp.float32)

    def body(i, carry):
        distance, farthest = carry
        slotf = (iota_s == i).astype(jnp.float32)
        cent_ref[...] += slotf * farthest.astype(jnp.float32)
        onehot = (iota == farthest)
        cx = jnp.sum(jnp.where(onehot, x, 0.0), axis=1, keepdims=True)
        cy = jnp.sum(jnp.where(onehot, y, 0.0), axis=1, keepdims=True)
        cz = jnp.sum(jnp.where(onehot, z, 0.0), axis=1, keepdims=True)
        nx_ref[...] += slotf * cx
        ny_ref[...] += slotf * cy
        nz_ref[...] += slotf * cz
        dx = x - cx
        dy = y - cy
        dz = z - cz
        dist = (dx * dx + dy * dy) + dz * dz
        distance = jnp.minimum(distance, dist)
        m = jnp.max(distance, axis=1, keepdims=True)
        farthest = jnp.min(jnp.where(distance == m, iota, N),
                           axis=1, keepdims=True)
        return distance, farthest

    jax.lax.fori_loop(
        0, S, body,
        (jnp.full((B, N), 1e10, jnp.float32), jnp.zeros((B, 1), jnp.int32)))


def _fps_pallas(xyz, npoint):
    B, N, _ = xyz.shape
    x = xyz[:, :, 0]
    y = xyz[:, :, 1]
    z = xyz[:, :, 2]
    sds = jax.ShapeDtypeStruct
    cent, nx, ny, nz = pl.pallas_call(
        _fps_body,
        out_shape=(sds((B, npoint), jnp.float32), sds((B, npoint), jnp.float32),
                   sds((B, npoint), jnp.float32), sds((B, npoint), jnp.float32)),
    )(x, y, z)
    return cent.astype(jnp.int32), jnp.stack([nx, ny, nz], axis=-1)


def _sa3_body(xyz_ref, pts_ref, a1, c1, a2, c2, a3, c3, out_ref):
    B, S, _ = xyz_ref.shape
    h = jnp.concatenate([xyz_ref[...], pts_ref[...]], axis=-1)
    h = h.reshape(B * S, h.shape[-1])
    h = jnp.maximum(jnp.dot(h, a1[...], preferred_element_type=jnp.float32)
                    + c1[...][0], 0.0)
    h = jnp.maximum(jnp.dot(h, a2[...], preferred_element_type=jnp.float32)
                    + c2[...][0], 0.0)
    h = jnp.maximum(jnp.dot(h, a3[...], preferred_element_type=jnp.float32)
                    + c3[...][0], 0.0)
    h = h.reshape(B, S, h.shape[-1])
    out_ref[...] = jnp.max(h, axis=1)


def _sa3_pallas(l2_xyz, l2_points, layers):
    B, S, _ = l2_xyz.shape
    a1, c1, a2, c2, a3, c3 = _fuse_mlp(layers)
    cout = a3.shape[1]
    return pl.pallas_call(
        _sa3_body,
        out_shape=jax.ShapeDtypeStruct((B, cout), jnp.float32),
    )(l2_xyz, l2_points, a1, c1[None], a2, c2[None], a3, c3[None])


def _fp3_body(pts_ref, glob_ref, a1, c1, a2, c2, out_ref):
    B, S, C = pts_ref.shape
    G = glob_ref.shape[-1]
    glob = jnp.broadcast_to(glob_ref[...][:, None, :], (B, S, G))
    h = jnp.concatenate([pts_ref[...], glob], axis=-1).reshape(B * S, C + G)
    h = jnp.maximum(jnp.dot(h, a1[...], preferred_element_type=jnp.float32)
                    + c1[...][0], 0.0)
    h = jnp.maximum(jnp.dot(h, a2[...], preferred_element_type=jnp.float32)
                    + c2[...][0], 0.0)
    out_ref[...] = h.reshape(B, S, h.shape[-1])


def _fp3_pallas(l2_points, l3_points, layers):
    B, S, _ = l2_points.shape
    a1, c1, a2, c2 = _fuse_mlp(layers)
    cout = a2.shape[1]
    return pl.pallas_call(
        _fp3_body,
        out_shape=jax.ShapeDtypeStruct((B, S, cout), jnp.float32),
    )(l2_points, l3_points, a1, c1[None], a2, c2[None])


def _top3_interp(d, pts2):
    R, S = d.shape
    iota = jax.lax.broadcasted_iota(jnp.int32, (R, S), 1)
    wmat = jnp.zeros((R, S), jnp.float32)
    recips = []
    onehots = []
    dd = d
    for _ in range(3):
        m = jnp.min(dd, axis=-1, keepdims=True)
        sel = (dd == m)
        first = jnp.min(jnp.where(sel, iota, S), axis=-1, keepdims=True)
        hit = (iota == first)
        recips.append(1.0 / (jnp.maximum(m, 0.0) + 1e-8))
        onehots.append(hit)
        dd = jnp.where(hit, _BIG, dd)
    norm = recips[0] + recips[1] + recips[2]
    for r, hit in zip(recips, onehots):
        wmat = wmat + jnp.where(hit, r / norm, 0.0)
    return jnp.dot(wmat, pts2, preferred_element_type=jnp.float32)


def _fp2_body(x1_ref, x2_ref, p1_ref, p2_ref, a1, c1, a2, c2, out_ref):
    x1 = x1_ref[0]
    x2 = x2_ref[0]
    d = (jnp.sum(x1 * x1, -1, keepdims=True)
         + jnp.sum(x2 * x2, -1)[None, :]
         - 2.0 * jax.lax.dot_general(x1, x2, (((1,), (1,)), ((), ())),
                                     preferred_element_type=jnp.float32))
    interp = _top3_interp(d, p2_ref[0])
    h = jnp.concatenate([p1_ref[0], interp], axis=-1)
    h = jnp.maximum(jnp.dot(h, a1[...], preferred_element_type=jnp.float32)
                    + c1[...][0], 0.0)
    h = jnp.maximum(jnp.dot(h, a2[...], preferred_element_type=jnp.float32)
                    + c2[...][0], 0.0)
    out_ref[0] = h


def _fp2_pallas(xyz1, xyz2, points1, points2, layers):
    B, N, _ = xyz1.shape
    S = xyz2.shape[1]
    C1 = points1.shape[-1]
    C2 = points2.shape[-1]
    a1, c1, a2, c2 = _fuse_mlp(layers)
    cout = a2.shape[1]
    return pl.pallas_call(
        _fp2_body,
        grid=(B,),
        in_specs=[
            pl.BlockSpec((1, N, 3), lambda b: (b, 0, 0)),
            pl.BlockSpec((1, S, 3), lambda b: (b, 0, 0)),
            pl.BlockSpec((1, N, C1), lambda b: (b, 0, 0)),
            pl.BlockSpec((1, S, C2), lambda b: (b, 0, 0)),
            pl.BlockSpec(a1.shape, lambda b: (0, 0)),
            pl.BlockSpec((1,) + c1.shape, lambda b: (0, 0)),
            pl.BlockSpec(a2.shape, lambda b: (0, 0)),
            pl.BlockSpec((1,) + c2.shape, lambda b: (0, 0)),
        ],
        out_specs=pl.BlockSpec((1, N, cout), lambda b: (b, 0, 0)),
        out_shape=jax.ShapeDtypeStruct((B, N, cout), jnp.float32),
    )(xyz1, xyz2, points1, points2, a1, c1[None], a2, c2[None])


def _fp1_head_body(x1_ref, x2_ref, p2_ref, a1, c1, a2, c2, ah, ch, af, cf,
                   out_ref):
    x1 = x1_ref[0]
    x2 = x2_ref[0]
    d = (jnp.sum(x1 * x1, -1, keepdims=True)
         + jnp.sum(x2 * x2, -1)[None, :]
         - 2.0 * jax.lax.dot_general(x1, x2, (((1,), (1,)), ((), ())),
                                     preferred_element_type=jnp.float32))
    interp = _top3_interp(d, p2_ref[0])
    h = jnp.concatenate([x1, interp], axis=-1)
    h = jnp.maximum(jnp.dot(h, a1[...], preferred_element_type=jnp.float32)
                    + c1[...][0], 0.0)
    h = jnp.maximum(jnp.dot(h, a2[...], preferred_element_type=jnp.float32)
                    + c2[...][0], 0.0)
    h = jnp.maximum(jnp.dot(h, ah[...], preferred_element_type=jnp.float32)
                    + ch[...][0], 0.0)
    out_ref[0] = jnp.dot(h, af[...], preferred_element_type=jnp.float32) \
        + cf[...][0]


def _fp1_head_pallas(xyz, l1_xyz, l1_up, fp1_layers, head_layers, conv2, coord):
    B, N, _ = xyz.shape
    S = l1_xyz.shape[1]
    C2 = l1_up.shape[-1]
    a1, c1, a2, c2 = _fuse_mlp(fp1_layers)
    ah, ch = _fuse_layer(head_layers[0])
    wf = jnp.concatenate([conv2['W'], coord['W']], axis=0)
    bf = jnp.concatenate([conv2['b'], coord['b']], axis=0)
    af = jnp.zeros((wf.shape[1], 128), jnp.float32).at[:, :6].set(wf.T)
    cf = jnp.zeros((128,), jnp.float32).at[:6].set(bf)
    RC = 1024
    out = pl.pallas_call(
        _fp1_head_body,
        grid=(B, N // RC),
        in_specs=[
            pl.BlockSpec((1, RC, 3), lambda b, r: (b, r, 0)),
            pl.BlockSpec((1, S, 3), lambda b, r: (b, 0, 0)),
            pl.BlockSpec((1, S, C2), lambda b, r: (b, 0, 0)),
            pl.BlockSpec(a1.shape, lambda b, r: (0, 0)),
            pl.BlockSpec((1,) + c1.shape, lambda b, r: (0, 0)),
            pl.BlockSpec(a2.shape, lambda b, r: (0, 0)),
            pl.BlockSpec((1,) + c2.shape, lambda b, r: (0, 0)),
            pl.BlockSpec(ah.shape, lambda b, r: (0, 0)),
            pl.BlockSpec((1,) + ch.shape, lambda b, r: (0, 0)),
            pl.BlockSpec(af.shape, lambda b, r: (0, 0)),
            pl.BlockSpec((1,) + cf.shape, lambda b, r: (0, 0)),
        ],
        out_specs=pl.BlockSpec((1, RC, 128), lambda b, r: (b, r, 0)),
        out_shape=jax.ShapeDtypeStruct((B, N, 128), jnp.float32),
    )(xyz, l1_xyz, l1_up, a1, c1[None], a2, c2[None], ah, ch[None], af,
      cf[None])
    return out[..., :3], out[..., 3:6]


def _mask_body(new_ref, xyz_ref, out_ref, *, radii):
    new = new_ref[0]
    xyz = xyz_ref[0]
    d = (jnp.sum(new * new, -1, keepdims=True)
         + jnp.sum(xyz * xyz, -1)[None, :]
         - 2.0 * lax.dot_general(new, xyz, (((1,), (1,)), ((), ())),
                                 preferred_element_type=jnp.float32))
    w = jnp.zeros(d.shape, jnp.int32)
    for bi, r in enumerate(radii):
        w = w + jnp.where(d > r * r, 0, 1 << bi)
    out_ref[0] = w


def _radius_masks(new_xyz, xyz, radii):
    B, S, _ = new_xyz.shape
    N = xyz.shape[1]
    R = min(S, 128)
    return pl.pallas_call(
        functools.partial(_mask_body, radii=radii),
        grid=(B, S // R),
        in_specs=[
            pl.BlockSpec((1, R, 3), lambda b, s: (b, s, 0)),
            pl.BlockSpec((1, N, 3), lambda b, s: (b, 0, 0)),
        ],
        out_specs=pl.BlockSpec((1, R, N), lambda b, s: (b, s, 0)),
        out_shape=jax.ShapeDtypeStruct((B, S, N), jnp.int32),
    )(new_xyz, xyz)


def _sc_info():
    info = plsc.get_sparse_core_info()
    return info.num_cores, info.num_subcores


def _sc_ball_compact(mw_flat, B, S, N, Ks):
    NC, NS = _sc_info()
    NW = NC * NS
    G = (B * S) // 16
    GPW = G // NW
    GPS = S // 16
    mesh = plsc.VectorSubcoreMesh(core_axis_name="c", subcore_axis_name="s")
    sds = jax.ShapeDtypeStruct
    scratch = [pltpu.VMEM((N * 16,), jnp.int32)]
    scratch += [pltpu.VMEM((16, K), jnp.int32) for K in Ks]

    @functools.partial(
        pl.kernel, mesh=mesh,
        out_type=tuple(sds((B * S, K), jnp.int32) for K in Ks),
        scratch_types=scratch,
        compiler_params=pltpu.CompilerParams(needs_layout_passes=False))
    def k(mw_hbm, *refs):
        outs = refs[:len(Ks)]
        mwv = refs[len(Ks)]
        bufs = refs[len(Ks) + 1:]
        wid = lax.axis_index("s") * NC + lax.axis_index("c")
        iota16 = lax.iota(jnp.int32, 16)
        zero16 = jnp.zeros((16,), jnp.int32)
        rowbase = iota16 * N

        def group_body(gi, _):
            g = wid * GPW + gi
            b = g // GPS
            pltpu.sync_copy(mw_hbm.at[pl.ds(g * 16 * N, 16 * N)], mwv)
            base_val = b * N

            def jbody(j, offs):
                mv = plsc.load_gather(mwv, [rowbase + j])
                val = zero16 + (base_val + j)
                new_offs = []
                for bi, K in enumerate(Ks):
                    m = (mv & (1 << bi)) != 0
                    can = m & (offs[bi] < K)
                    col = jnp.minimum(offs[bi], K - 1)
                    plsc.store_scatter(bufs[bi], [iota16, col], val, mask=can)
                    new_offs.append(offs[bi] + jnp.where(can, 1, 0))
                return tuple(new_offs)

            offs = lax.fori_loop(0, N, jbody,
                                 tuple(zero16 for _ in Ks))
            for bi, K in enumerate(Ks):
                first0 = plsc.load_gather(bufs[bi], [iota16, zero16])

                def fbody(kk, _, bi=bi, K=K, first0=first0):
                    need = offs[bi] <= kk
                    plsc.store_scatter(bufs[bi], [iota16, zero16 + kk],
                                       first0, mask=need)
                    return 0

                lax.fori_loop(1, K, fbody, 0)
                pltpu.sync_copy(bufs[bi], outs[bi].at[pl.ds(g * 16, 16)])
            return 0

        lax.fori_loop(0, GPW, group_body, 0)

    return k(mw_flat)


def _sc_gather_rows(table, idx):
    NC, NS = _sc_info()
    NW = NC * NS
    M = idx.shape[0]
    D = table.shape[1]
    m_per_w = M // NW
    ch = m_per_w
    while ch * D * 4 > 260000 or ch > 4096:
        ch //= 2
    nch = m_per_w // ch
    mesh = plsc.VectorSubcoreMesh(core_axis_name="c", subcore_axis_name="s")

    @functools.partial(
        pl.kernel, mesh=mesh,
        out_type=jax.ShapeDtypeStruct((M, D), jnp.float32),
        scratch_types=[pltpu.VMEM((ch,), jnp.int32),
                       pltpu.VMEM((ch, D), jnp.float32),
                       pltpu.SemaphoreType.DMA],
        compiler_params=pltpu.CompilerParams(needs_layout_passes=False,
                                             use_tc_tiling_on_sc=False))
    def k(table_hbm, idx_hbm, out_hbm, idx_v, rows_v, sem):
        wid = lax.axis_index("s") * NC + lax.axis_index("c")
        base = wid * m_per_w

        def body(c, _):
            off = base + c * ch
            pltpu.sync_copy(idx_hbm.at[pl.ds(off, ch)], idx_v)
            pltpu.async_copy(table_hbm.at[idx_v], rows_v, sem).wait()
            pltpu.sync_copy(rows_v, out_hbm.at[pl.ds(off, ch)])
            return 0

        lax.fori_loop(0, nch, body, 0)

    return k(table, idx)


def _sa1_mlp_body(g_ref, new_ref, a1, c1, a2, c2, a3, c3, out_ref, *, K):
    RK = g_ref.shape[0]
    R = RK // K
    new = new_ref[...]
    t = g_ref[...].reshape(R, K, 16)[:, :, :3] - new[:, None, :]
    t = t.reshape(RK, 3)
    a = a1[...]
    h = (t[:, 0:1] * a[0] + t[:, 1:2] * a[1] + t[:, 2:3] * a[2]
         + c1[...][0])
    h = jnp.maximum(h, 0.0)
    h = jnp.maximum(jnp.dot(h, a2[...], preferred_element_type=jnp.float32)
                    + c2[...][0], 0.0)
    h = jnp.maximum(jnp.dot(h, a3[...], preferred_element_type=jnp.float32)
                    + c3[...][0], 0.0)
    out_ref[...] = jnp.max(h.reshape(R, K, h.shape[-1]), axis=1)


def _sa1_branch_mlp(gathered, new_flat, K, layers):
    BS = new_flat.shape[0]
    a1, c1, a2, c2, a3, c3 = _fuse_mlp(layers)
    cout = a3.shape[1]
    R = 64
    return pl.pallas_call(
        functools.partial(_sa1_mlp_body, K=K),
        grid=(BS // R,),
        in_specs=[
            pl.BlockSpec((R * K, 16), lambda i: (i, 0)),
            pl.BlockSpec((R, 3), lambda i: (i, 0)),
            pl.BlockSpec(a1.shape, lambda i: (0, 0)),
            pl.BlockSpec((1,) + c1.shape, lambda i: (0, 0)),
            pl.BlockSpec(a2.shape, lambda i: (0, 0)),
            pl.BlockSpec((1,) + c2.shape, lambda i: (0, 0)),
            pl.BlockSpec(a3.shape, lambda i: (0, 0)),
            pl.BlockSpec((1,) + c3.shape, lambda i: (0, 0)),
        ],
        out_specs=pl.BlockSpec((R, cout), lambda i: (i, 0)),
        out_shape=jax.ShapeDtypeStruct((BS, cout), jnp.float32),
    )(gathered, new_flat, a1, c1[None], a2, c2[None], a3, c3[None])


def _sa2_tables_body(pts_ref, xyz_ref, a1_b1, c1_b1, a1_b2, c1_b2, t1_ref,
                     t2_ref):
    BN = t1_ref.shape[0]
    h = jnp.concatenate([pts_ref[...], xyz_ref[...]], axis=-1)
    h = h.reshape(BN, h.shape[-1])
    t1_ref[...] = jnp.dot(h, a1_b1[...], preferred_element_type=jnp.float32) \
        + c1_b1[...][0]
    t2_ref[...] = jnp.dot(h, a1_b2[...], preferred_element_type=jnp.float32) \
        + c1_b2[...][0]


def _sa2_tables(points, xyz, a1_b1, c1_b1, a1_b2, c1_b2):
    B, N, CF = points.shape
    cout = a1_b1.shape[1]
    sds = jax.ShapeDtypeStruct
    return pl.pallas_call(
        _sa2_tables_body,
        out_shape=(sds((B * N, cout), jnp.float32),
                   sds((B * N, cout), jnp.float32)),
    )(points, xyz, a1_b1, c1_b1[None], a1_b2, c1_b2[None])


def _sa2_mlp_body(g_ref, new_ref, a1x, a2, c2, a3, c3, out_ref, *, K):
    RK = g_ref.shape[0]
    R = RK // K
    ca = jnp.dot(new_ref[...], a1x[...], preferred_element_type=jnp.float32)
    h = g_ref[...].reshape(R, K, g_ref.shape[-1]) - ca[:, None, :]
    h = jnp.maximum(h.reshape(RK, h.shape[-1]), 0.0)
    h = jnp.maximum(jnp.dot(h, a2[...], preferred_element_type=jnp.float32)
                    + c2[...][0], 0.0)
    h = jnp.maximum(jnp.dot(h, a3[...], preferred_element_type=jnp.float32)
                    + c3[...][0], 0.0)
    out_ref[...] = jnp.max(h.reshape(R, K, h.shape[-1]), axis=1)


def _sa2_branch_mlp(gathered, new_flat, K, a1x, a2, c2, a3, c3):
    BS = new_flat.shape[0]
    C1 = gathered.shape[1]
    cout = a3.shape[1]
    R = 32
    return pl.pallas_call(
        functools.partial(_sa2_mlp_body, K=K),
        grid=(BS // R,),
        in_specs=[
            pl.BlockSpec((R * K, C1), lambda i: (i, 0)),
            pl.BlockSpec((R, 3), lambda i: (i, 0)),
            pl.BlockSpec(a1x.shape, lambda i: (0, 0)),
            pl.BlockSpec(a2.shape, lambda i: (0, 0)),
            pl.BlockSpec((1,) + c2.shape, lambda i: (0, 0)),
            pl.BlockSpec(a3.shape, lambda i: (0, 0)),
            pl.BlockSpec((1,) + c3.shape, lambda i: (0, 0)),
        ],
        out_specs=pl.BlockSpec((R, cout), lambda i: (i, 0)),
        out_shape=jax.ShapeDtypeStruct((BS, cout), jnp.float32),
    )(gathered, new_flat, a1x, a2, c2[None], a3, c3[None])


def _sa_msg_sc(xyz, points, new_xyz, radii, Ks, branch_params):
    B, N, _ = xyz.shape
    S = new_xyz.shape[1]
    BS = B * S
    mw = _radius_masks(new_xyz, xyz, radii)
    gidx = _sc_ball_compact(mw.reshape(B * S * N), B, S, N, Ks)
    new_flat = new_xyz.reshape(BS, 3)
    Ksum = sum(Ks)
    outs = []
    if points is None:
        all_idx = jnp.concatenate(gidx, axis=1).reshape(-1)
        table = jnp.pad(xyz.reshape(B * N, 3), ((0, 0), (0, 13)))
        rows = _sc_gather_rows(table, all_idx)
        rows3 = rows.reshape(BS, Ksum, rows.shape[-1])
        col = 0
        for K, layers in zip(Ks, branch_params):
            gr = rows3[:, col:col + K, :].reshape(BS * K, rows.shape[-1])
            col += K
            outs.append(_sa1_branch_mlp(gr, new_flat, K, layers))
    else:
        CF = points.shape[-1]
        fused = [_fuse_mlp(layers) for layers in branch_params]
        t1, t2 = _sa2_tables(points, xyz, fused[0][0], fused[0][1],
                             fused[1][0], fused[1][1])
        all_idx = jnp.concatenate([gidx[0], gidx[1] + B * N],
                                  axis=1).reshape(-1)
        table = jnp.concatenate([t1, t2], axis=0)
        rows = _sc_gather_rows(table, all_idx)
        rows3 = rows.reshape(BS, Ksum, rows.shape[-1])
        col = 0
        for K, fl in zip(Ks, fused):
            gr = rows3[:, col:col + K, :].reshape(BS * K, rows.shape[-1])
            col += K
            a1x = fl[0][CF:CF + 3]
            outs.append(_sa2_branch_mlp(gr, new_flat, K, a1x,
                                        fl[2], fl[3], fl[4], fl[5]))
    return jnp.concatenate(outs, -1).reshape(B, S, -1)


def kernel(xyz, params):
    _, l1_xyz = _fps_pallas(xyz, 512)
    l1_points = _sa_msg_sc(xyz, None, l1_xyz, (0.1, 0.2, 0.4),
                           (32, 64, 128), params['sa1'])
    _, l2_xyz = _fps_pallas(l1_xyz, 128)
    l2_points = _sa_msg_sc(l1_xyz, l1_points, l2_xyz, (0.4, 0.8),
                           (64, 128), params['sa2'])
    l3_points = _sa3_pallas(l2_xyz, l2_points, params['sa3'])
    l2_up = _fp3_pallas(l2_points, l3_points, params['fp3'])
    l1_up = _fp2_pallas(l1_xyz, l2_xyz, l1_points, l2_up, params['fp2'])
    seg_logits, coords = _fp1_head_pallas(xyz, l1_xyz, l1_up, params['fp1'],
                                          params['head'], params['conv2'],
                                          params['coord'])
    return seg_logits, coords


# speedup vs baseline: 1.0831x; 1.0831x over previous
import functools

import jax
import jax.numpy as jnp
import numpy as np
from jax import lax
from jax.experimental import pallas as pl
from jax.experimental.pallas import tpu as pltpu
from jax.experimental.pallas import tpu_sc as plsc

_BN_EPS = 1e-5
_BIG = 1e30


def _fuse_layer(layer):
    s = layer['gamma'] / jnp.sqrt(1.0 + _BN_EPS)
    A = layer['W'].T * s[None, :]
    c = layer['b'] * s + layer['beta']
    return A, c


def _fuse_mlp(layers):
    out = []
    for l in layers:
        out.extend(_fuse_layer(l))
    return out


def _square_distance(src, dst):
    return (jnp.sum(src ** 2, -1)[:, :, None] + jnp.sum(dst ** 2, -1)[:, None, :]
            - 2.0 * jnp.einsum('bnc,bmc->bnm', src, dst))


def _index_points(points, idx):
    return jax.vmap(lambda p, i: p[i])(points, idx)


def _farthest_point_sample(xyz, npoint):
    B, N, _ = xyz.shape

    def body(i, carry):
        centroids, distance, farthest = carry
        centroids = centroids.at[:, i].set(farthest)
        centroid = jnp.take_along_axis(xyz, farthest[:, None, None], axis=1)
        dist = jnp.sum((xyz - centroid) ** 2, -1)
        distance = jnp.minimum(distance, dist)
        farthest = jnp.argmax(distance, -1).astype(jnp.int32)
        return centroids, distance, farthest

    centroids = jnp.zeros((B, npoint), jnp.int32)
    distance = jnp.full((B, N), 1e10, jnp.float32)
    farthest = jnp.zeros((B,), jnp.int32)
    centroids, _, _ = jax.lax.fori_loop(0, npoint, body,
                                        (centroids, distance, farthest))
    return centroids


def _query_ball_point(radius, nsample, xyz, new_xyz):
    B, N, _ = xyz.shape
    S = new_xyz.shape[1]
    sqrdists = _square_distance(new_xyz, xyz)
    group_idx = jnp.broadcast_to(
        jnp.arange(N, dtype=jnp.int32)[None, None, :], (B, S, N))
    group_idx = jnp.where(sqrdists > radius ** 2, N, group_idx)
    group_idx = jnp.sort(group_idx, axis=-1)[:, :, :nsample]
    group_first = jnp.broadcast_to(group_idx[:, :, :1], group_idx.shape)
    group_idx = jnp.where(group_idx == N, group_first, group_idx)
    return group_idx


def _apply_layer_xla(x, layer):
    h = jnp.einsum('...i,oi->...o', x, layer['W']) + layer['b']
    h = h * (layer['gamma'] / jnp.sqrt(1.0 + _BN_EPS)) + layer['beta']
    return jax.nn.relu(h)


def _sa_msg_xla(xyz, points, npoint, radius_list, nsample_list, branch_params,
                new_xyz=None):
    if new_xyz is None:
        fps_idx = _farthest_point_sample(xyz, npoint)
        new_xyz = _index_points(xyz, fps_idx)
    outs = []
    for radius, K, layers in zip(radius_list, nsample_list, branch_params):
        group_idx = _query_ball_point(radius, K, xyz, new_xyz)
        grouped_xyz = _index_points(xyz, group_idx) - new_xyz[:, :, None, :]
        if points is not None:
            h = jnp.concatenate([_index_points(points, group_idx), grouped_xyz], -1)
        else:
            h = grouped_xyz
        for layer in layers:
            h = _apply_layer_xla(h, layer)
        outs.append(jnp.max(h, axis=2))
    return new_xyz, jnp.concatenate(outs, -1)


def _fps_body(x_ref, y_ref, z_ref, cent_ref, nx_ref, ny_ref, nz_ref):
    B, N = x_ref.shape
    S = cent_ref.shape[1]
    x = x_ref[...]
    y = y_ref[...]
    z = z_ref[...]
    iota = jax.lax.broadcasted_iota(jnp.int32, (B, N), 1)
    iota_s = jax.lax.broadcasted_iota(jnp.int32, (B, S), 1)

    cent_ref[...] = jnp.zeros((B, S), jnp.float32)
    nx_ref[...] = jnp.zeros((B, S), jnp.float32)
    ny_ref[...] = jnp.zeros((B, S), jnp.float32)
    nz_ref[...] = jnp.zeros((B, S), jnp.float32)

    def body(i, carry):
        distance, farthest = carry
        slotf = (iota_s == i).astype(jnp.float32)
        cent_ref[...] += slotf * farthest.astype(jnp.float32)
        onehot = (iota == farthest)
        cx = jnp.sum(jnp.where(onehot, x, 0.0), axis=1, keepdims=True)
        cy = jnp.sum(jnp.where(onehot, y, 0.0), axis=1, keepdims=True)
        cz = jnp.sum(jnp.where(onehot, z, 0.0), axis=1, keepdims=True)
        nx_ref[...] += slotf * cx
        ny_ref[...] += slotf * cy
        nz_ref[...] += slotf * cz
        dx = x - cx
        dy = y - cy
        dz = z - cz
        dist = (dx * dx + dy * dy) + dz * dz
        distance = jnp.minimum(distance, dist)
        m = jnp.max(distance, axis=1, keepdims=True)
        farthest = jnp.min(jnp.where(distance == m, iota, N),
                           axis=1, keepdims=True)
        return distance, farthest

    jax.lax.fori_loop(
        0, S, body,
        (jnp.full((B, N), 1e10, jnp.float32), jnp.zeros((B, 1), jnp.int32)))


def _fps_pallas(xyz, npoint):
    B, N, _ = xyz.shape
    x = xyz[:, :, 0]
    y = xyz[:, :, 1]
    z = xyz[:, :, 2]
    sds = jax.ShapeDtypeStruct
    cent, nx, ny, nz = pl.pallas_call(
        _fps_body,
        out_shape=(sds((B, npoint), jnp.float32), sds((B, npoint), jnp.float32),
                   sds((B, npoint), jnp.float32), sds((B, npoint), jnp.float32)),
    )(x, y, z)
    return cent.astype(jnp.int32), jnp.stack([nx, ny, nz], axis=-1)


def _sa3_body(xyz_ref, pts_ref, a1, c1, a2, c2, a3, c3, out_ref):
    B, S, _ = xyz_ref.shape
    h = jnp.concatenate([xyz_ref[...], pts_ref[...]], axis=-1)
    h = h.reshape(B * S, h.shape[-1])
    h = jnp.maximum(jnp.dot(h, a1[...], preferred_element_type=jnp.float32)
                    + c1[...][0], 0.0)
    h = jnp.maximum(jnp.dot(h, a2[...], preferred_element_type=jnp.float32)
                    + c2[...][0], 0.0)
    h = jnp.maximum(jnp.dot(h, a3[...], preferred_element_type=jnp.float32)
                    + c3[...][0], 0.0)
    h = h.reshape(B, S, h.shape[-1])
    out_ref[...] = jnp.max(h, axis=1)


def _sa3_pallas(l2_xyz, l2_points, layers):
    B, S, _ = l2_xyz.shape
    a1, c1, a2, c2, a3, c3 = _fuse_mlp(layers)
    cout = a3.shape[1]
    return pl.pallas_call(
        _sa3_body,
        out_shape=jax.ShapeDtypeStruct((B, cout), jnp.float32),
    )(l2_xyz, l2_points, a1, c1[None], a2, c2[None], a3, c3[None])


def _fp3_body(pts_ref, glob_ref, a1, c1, a2, c2, out_ref):
    B, S, C = pts_ref.shape
    G = glob_ref.shape[-1]
    glob = jnp.broadcast_to(glob_ref[...][:, None, :], (B, S, G))
    h = jnp.concatenate([pts_ref[...], glob], axis=-1).reshape(B * S, C + G)
    h = jnp.maximum(jnp.dot(h, a1[...], preferred_element_type=jnp.float32)
                    + c1[...][0], 0.0)
    h = jnp.maximum(jnp.dot(h, a2[...], preferred_element_type=jnp.float32)
                    + c2[...][0], 0.0)
    out_ref[...] = h.reshape(B, S, h.shape[-1])


def _fp3_pallas(l2_points, l3_points, layers):
    B, S, _ = l2_points.shape
    a1, c1, a2, c2 = _fuse_mlp(layers)
    cout = a2.shape[1]
    return pl.pallas_call(
        _fp3_body,
        out_shape=jax.ShapeDtypeStruct((B, S, cout), jnp.float32),
    )(l2_points, l3_points, a1, c1[None], a2, c2[None])


def _top3_interp(d, pts2):
    R, S = d.shape
    iota = jax.lax.broadcasted_iota(jnp.int32, (R, S), 1)
    wmat = jnp.zeros((R, S), jnp.float32)
    recips = []
    onehots = []
    dd = d
    for _ in range(3):
        m = jnp.min(dd, axis=-1, keepdims=True)
        sel = (dd == m)
        first = jnp.min(jnp.where(sel, iota, S), axis=-1, keepdims=True)
        hit = (iota == first)
        recips.append(1.0 / (jnp.maximum(m, 0.0) + 1e-8))
        onehots.append(hit)
        dd = jnp.where(hit, _BIG, dd)
    norm = recips[0] + recips[1] + recips[2]
    for r, hit in zip(recips, onehots):
        wmat = wmat + jnp.where(hit, r / norm, 0.0)
    return jnp.dot(wmat, pts2, preferred_element_type=jnp.float32)


def _fp2_body(x1_ref, x2_ref, p1_ref, p2_ref, a1, c1, a2, c2, out_ref):
    x1 = x1_ref[0]
    x2 = x2_ref[0]
    d = (jnp.sum(x1 * x1, -1, keepdims=True)
         + jnp.sum(x2 * x2, -1)[None, :]
         - 2.0 * jax.lax.dot_general(x1, x2, (((1,), (1,)), ((), ())),
                                     preferred_element_type=jnp.float32))
    interp = _top3_interp(d, p2_ref[0])
    h = jnp.concatenate([p1_ref[0], interp], axis=-1)
    h = jnp.maximum(jnp.dot(h, a1[...], preferred_element_type=jnp.float32)
                    + c1[...][0], 0.0)
    h = jnp.maximum(jnp.dot(h, a2[...], preferred_element_type=jnp.float32)
                    + c2[...][0], 0.0)
    out_ref[0] = h


def _fp2_pallas(xyz1, xyz2, points1, points2, layers):
    B, N, _ = xyz1.shape
    S = xyz2.shape[1]
    C1 = points1.shape[-1]
    C2 = points2.shape[-1]
    a1, c1, a2, c2 = _fuse_mlp(layers)
    cout = a2.shape[1]
    return pl.pallas_call(
        _fp2_body,
        grid=(B,),
        in_specs=[
            pl.BlockSpec((1, N, 3), lambda b: (b, 0, 0)),
            pl.BlockSpec((1, S, 3), lambda b: (b, 0, 0)),
            pl.BlockSpec((1, N, C1), lambda b: (b, 0, 0)),
            pl.BlockSpec((1, S, C2), lambda b: (b, 0, 0)),
            pl.BlockSpec(a1.shape, lambda b: (0, 0)),
            pl.BlockSpec((1,) + c1.shape, lambda b: (0, 0)),
            pl.BlockSpec(a2.shape, lambda b: (0, 0)),
            pl.BlockSpec((1,) + c2.shape, lambda b: (0, 0)),
        ],
        out_specs=pl.BlockSpec((1, N, cout), lambda b: (b, 0, 0)),
        out_shape=jax.ShapeDtypeStruct((B, N, cout), jnp.float32),
    )(xyz1, xyz2, points1, points2, a1, c1[None], a2, c2[None])


def _fp1_head_body(x1_ref, x2_ref, p2_ref, a1, c1, a2, c2, ah, ch, af, cf,
                   out_ref):
    x1 = x1_ref[0]
    x2 = x2_ref[0]
    d = (jnp.sum(x1 * x1, -1, keepdims=True)
         + jnp.sum(x2 * x2, -1)[None, :]
         - 2.0 * jax.lax.dot_general(x1, x2, (((1,), (1,)), ((), ())),
                                     preferred_element_type=jnp.float32))
    interp = _top3_interp(d, p2_ref[0])
    h = jnp.concatenate([x1, interp], axis=-1)
    h = jnp.maximum(jnp.dot(h, a1[...], preferred_element_type=jnp.float32)
                    + c1[...][0], 0.0)
    h = jnp.maximum(jnp.dot(h, a2[...], preferred_element_type=jnp.float32)
                    + c2[...][0], 0.0)
    h = jnp.maximum(jnp.dot(h, ah[...], preferred_element_type=jnp.float32)
                    + ch[...][0], 0.0)
    out_ref[0] = jnp.dot(h, af[...], preferred_element_type=jnp.float32) \
        + cf[...][0]


def _fp1_head_pallas(xyz, l1_xyz, l1_up, fp1_layers, head_layers, conv2, coord):
    B, N, _ = xyz.shape
    S = l1_xyz.shape[1]
    C2 = l1_up.shape[-1]
    a1, c1, a2, c2 = _fuse_mlp(fp1_layers)
    ah, ch = _fuse_layer(head_layers[0])
    wf = jnp.concatenate([conv2['W'], coord['W']], axis=0)
    bf = jnp.concatenate([conv2['b'], coord['b']], axis=0)
    af = jnp.zeros((wf.shape[1], 128), jnp.float32).at[:, :6].set(wf.T)
    cf = jnp.zeros((128,), jnp.float32).at[:6].set(bf)
    RC = 1024
    out = pl.pallas_call(
        _fp1_head_body,
        grid=(B, N // RC),
        in_specs=[
            pl.BlockSpec((1, RC, 3), lambda b, r: (b, r, 0)),
            pl.BlockSpec((1, S, 3), lambda b, r: (b, 0, 0)),
            pl.BlockSpec((1, S, C2), lambda b, r: (b, 0, 0)),
            pl.BlockSpec(a1.shape, lambda b, r: (0, 0)),
            pl.BlockSpec((1,) + c1.shape, lambda b, r: (0, 0)),
            pl.BlockSpec(a2.shape, lambda b, r: (0, 0)),
            pl.BlockSpec((1,) + c2.shape, lambda b, r: (0, 0)),
            pl.BlockSpec(ah.shape, lambda b, r: (0, 0)),
            pl.BlockSpec((1,) + ch.shape, lambda b, r: (0, 0)),
            pl.BlockSpec(af.shape, lambda b, r: (0, 0)),
            pl.BlockSpec((1,) + cf.shape, lambda b, r: (0, 0)),
        ],
        out_specs=pl.BlockSpec((1, RC, 128), lambda b, r: (b, r, 0)),
        out_shape=jax.ShapeDtypeStruct((B, N, 128), jnp.float32),
    )(xyz, l1_xyz, l1_up, a1, c1[None], a2, c2[None], ah, ch[None], af,
      cf[None])
    return out[..., :3], out[..., 3:6]


def _mask_body(new_ref, xyz_ref, out_ref, *, radii):
    new = new_ref[0]
    xyz = xyz_ref[0]
    d = (jnp.sum(new * new, -1, keepdims=True)
         + jnp.sum(xyz * xyz, -1)[None, :]
         - 2.0 * lax.dot_general(new, xyz, (((1,), (1,)), ((), ())),
                                 preferred_element_type=jnp.float32))
    w = jnp.zeros(d.shape, jnp.int32)
    for bi, r in enumerate(radii):
        w = w + jnp.where(d > r * r, 0, 1 << bi)
    out_ref[0] = w


def _radius_masks(new_xyz, xyz, radii):
    B, S, _ = new_xyz.shape
    N = xyz.shape[1]
    R = min(S, 128)
    return pl.pallas_call(
        functools.partial(_mask_body, radii=radii),
        grid=(B, S // R),
        in_specs=[
            pl.BlockSpec((1, R, 3), lambda b, s: (b, s, 0)),
            pl.BlockSpec((1, N, 3), lambda b, s: (b, 0, 0)),
        ],
        out_specs=pl.BlockSpec((1, R, N), lambda b, s: (b, s, 0)),
        out_shape=jax.ShapeDtypeStruct((B, S, N), jnp.int32),
    )(new_xyz, xyz)


def _sc_info():
    info = plsc.get_sparse_core_info()
    return info.num_cores, info.num_subcores


def _sc_ball_compact(mw_flat, B, S, N, Ks):
    NC, NS = _sc_info()
    NW = NC * NS
    G = (B * S) // 16
    GPW = G // NW
    GPS = S // 16
    mesh = plsc.VectorSubcoreMesh(core_axis_name="c", subcore_axis_name="s")
    sds = jax.ShapeDtypeStruct
    scratch = [pltpu.VMEM((N * 16,), jnp.int32)]
    scratch += [pltpu.VMEM((16, K), jnp.int32) for K in Ks]

    @functools.partial(
        pl.kernel, mesh=mesh,
        out_type=tuple(sds((B * S, K), jnp.int32) for K in Ks),
        scratch_types=scratch,
        compiler_params=pltpu.CompilerParams(needs_layout_passes=False))
    def k(mw_hbm, *refs):
        outs = refs[:len(Ks)]
        mwv = refs[len(Ks)]
        bufs = refs[len(Ks) + 1:]
        wid = lax.axis_index("s") * NC + lax.axis_index("c")
        iota16 = lax.iota(jnp.int32, 16)
        zero16 = jnp.zeros((16,), jnp.int32)
        rowbase = iota16 * N

        def group_body(gi, _):
            g = wid * GPW + gi
            b = g // GPS
            pltpu.sync_copy(mw_hbm.at[pl.ds(g * 16 * N, 16 * N)], mwv)
            base_val = b * N

            def jbody(j, offs):
                mv = plsc.load_gather(mwv, [rowbase + j])
                val = zero16 + (base_val + j)
                new_offs = []
                for bi, K in enumerate(Ks):
                    m = (mv & (1 << bi)) != 0
                    can = m & (offs[bi] < K)
                    col = jnp.minimum(offs[bi], K - 1)
                    plsc.store_scatter(bufs[bi], [iota16, col], val, mask=can)
                    new_offs.append(offs[bi] + jnp.where(can, 1, 0))
                return tuple(new_offs)

            offs = lax.fori_loop(0, N, jbody,
                                 tuple(zero16 for _ in Ks))
            for bi, K in enumerate(Ks):
                first0 = plsc.load_gather(bufs[bi], [iota16, zero16])

                def fbody(kk, _, bi=bi, K=K, first0=first0):
                    need = offs[bi] <= kk
                    plsc.store_scatter(bufs[bi], [iota16, zero16 + kk],
                                       first0, mask=need)
                    return 0

                lax.fori_loop(1, K, fbody, 0)
                pltpu.sync_copy(bufs[bi], outs[bi].at[pl.ds(g * 16, 16)])
            return 0

        lax.fori_loop(0, GPW, group_body, 0)

    return k(mw_flat)


def _sc_gather_rows(table, idx):
    NC, NS = _sc_info()
    NW = NC * NS
    M = idx.shape[0]
    D = table.shape[1]
    m_per_w = M // NW
    ch = m_per_w
    while ch * D * 4 > 260000 or ch > 4096:
        ch //= 2
    nch = m_per_w // ch
    mesh = plsc.VectorSubcoreMesh(core_axis_name="c", subcore_axis_name="s")

    @functools.partial(
        pl.kernel, mesh=mesh,
        out_type=jax.ShapeDtypeStruct((M, D), jnp.float32),
        scratch_types=[pltpu.VMEM((ch,), jnp.int32),
                       pltpu.VMEM((ch, D), jnp.float32),
                       pltpu.SemaphoreType.DMA],
        compiler_params=pltpu.CompilerParams(needs_layout_passes=False,
                                             use_tc_tiling_on_sc=False))
    def k(table_hbm, idx_hbm, out_hbm, idx_v, rows_v, sem):
        wid = lax.axis_index("s") * NC + lax.axis_index("c")
        base = wid * m_per_w

        def body(c, _):
            off = base + c * ch
            pltpu.sync_copy(idx_hbm.at[pl.ds(off, ch)], idx_v)
            pltpu.async_copy(table_hbm.at[idx_v], rows_v, sem).wait()
            pltpu.sync_copy(rows_v, out_hbm.at[pl.ds(off, ch)])
            return 0

        lax.fori_loop(0, nch, body, 0)

    return k(table, idx)


def _sa1_mlp_body(g_ref, new_ref, a1, c1, a2, c2, a3, c3, out_ref, *, K):
    RK = g_ref.shape[0]
    R = RK // K
    new = new_ref[...]
    t = g_ref[...].reshape(R, K, 16)[:, :, :3] - new[:, None, :]
    h = t.reshape(RK, 3)
    h = jnp.maximum(jnp.dot(h, a1[...], preferred_element_type=jnp.float32)
                    + c1[...][0], 0.0)
    h = jnp.maximum(jnp.dot(h, a2[...], preferred_element_type=jnp.float32)
                    + c2[...][0], 0.0)
    h = jnp.maximum(jnp.dot(h, a3[...], preferred_element_type=jnp.float32)
                    + c3[...][0], 0.0)
    out_ref[...] = jnp.max(h.reshape(R, K, h.shape[-1]), axis=1)


def _sa1_branch_mlp(gathered, new_flat, K, layers):
    BS = new_flat.shape[0]
    a1, c1, a2, c2, a3, c3 = _fuse_mlp(layers)
    cout = a3.shape[1]
    R = 64
    return pl.pallas_call(
        functools.partial(_sa1_mlp_body, K=K),
        grid=(BS // R,),
        in_specs=[
            pl.BlockSpec((R * K, 16), lambda i: (i, 0)),
            pl.BlockSpec((R, 3), lambda i: (i, 0)),
            pl.BlockSpec(a1.shape, lambda i: (0, 0)),
            pl.BlockSpec((1,) + c1.shape, lambda i: (0, 0)),
            pl.BlockSpec(a2.shape, lambda i: (0, 0)),
            pl.BlockSpec((1,) + c2.shape, lambda i: (0, 0)),
            pl.BlockSpec(a3.shape, lambda i: (0, 0)),
            pl.BlockSpec((1,) + c3.shape, lambda i: (0, 0)),
        ],
        out_specs=pl.BlockSpec((R, cout), lambda i: (i, 0)),
        out_shape=jax.ShapeDtypeStruct((BS, cout), jnp.float32),
    )(gathered, new_flat, a1, c1[None], a2, c2[None], a3, c3[None])


def _sa2_tables_body(pts_ref, xyz_ref, a1_b1, c1_b1, a1_b2, c1_b2, t1_ref,
                     t2_ref):
    BN = t1_ref.shape[0]
    h = jnp.concatenate([pts_ref[...], xyz_ref[...]], axis=-1)
    h = h.reshape(BN, h.shape[-1])
    t1_ref[...] = jnp.dot(h, a1_b1[...], preferred_element_type=jnp.float32) \
        + c1_b1[...][0]
    t2_ref[...] = jnp.dot(h, a1_b2[...], preferred_element_type=jnp.float32) \
        + c1_b2[...][0]


def _sa2_tables(points, xyz, a1_b1, c1_b1, a1_b2, c1_b2):
    B, N, CF = points.shape
    cout = a1_b1.shape[1]
    sds = jax.ShapeDtypeStruct
    return pl.pallas_call(
        _sa2_tables_body,
        out_shape=(sds((B * N, cout), jnp.float32),
                   sds((B * N, cout), jnp.float32)),
    )(points, xyz, a1_b1, c1_b1[None], a1_b2, c1_b2[None])


def _sa2_mlp_body(g_ref, new_ref, a1x, a2, c2, a3, c3, out_ref, *, K):
    RK = g_ref.shape[0]
    R = RK // K
    ca = jnp.dot(new_ref[...], a1x[...], preferred_element_type=jnp.float32)
    h = g_ref[...].reshape(R, K, g_ref.shape[-1]) - ca[:, None, :]
    h = jnp.maximum(h.reshape(RK, h.shape[-1]), 0.0)
    h = jnp.maximum(jnp.dot(h, a2[...], preferred_element_type=jnp.float32)
                    + c2[...][0], 0.0)
    h = jnp.maximum(jnp.dot(h, a3[...], preferred_element_type=jnp.float32)
                    + c3[...][0], 0.0)
    out_ref[...] = jnp.max(h.reshape(R, K, h.shape[-1]), axis=1)


def _sa2_branch_mlp(gathered, new_flat, K, a1x, a2, c2, a3, c3):
    BS = new_flat.shape[0]
    C1 = gathered.shape[1]
    cout = a3.shape[1]
    R = 32
    return pl.pallas_call(
        functools.partial(_sa2_mlp_body, K=K),
        grid=(BS // R,),
        in_specs=[
            pl.BlockSpec((R * K, C1), lambda i: (i, 0)),
            pl.BlockSpec((R, 3), lambda i: (i, 0)),
            pl.BlockSpec(a1x.shape, lambda i: (0, 0)),
            pl.BlockSpec(a2.shape, lambda i: (0, 0)),
            pl.BlockSpec((1,) + c2.shape, lambda i: (0, 0)),
            pl.BlockSpec(a3.shape, lambda i: (0, 0)),
            pl.BlockSpec((1,) + c3.shape, lambda i: (0, 0)),
        ],
        out_specs=pl.BlockSpec((R, cout), lambda i: (i, 0)),
        out_shape=jax.ShapeDtypeStruct((BS, cout), jnp.float32),
    )(gathered, new_flat, a1x, a2, c2[None], a3, c3[None])


def _sa_msg_sc(xyz, points, new_xyz, radii, Ks, branch_params):
    B, N, _ = xyz.shape
    S = new_xyz.shape[1]
    BS = B * S
    mw = _radius_masks(new_xyz, xyz, radii)
    gidx = _sc_ball_compact(mw.reshape(B * S * N), B, S, N, Ks)
    new_flat = new_xyz.reshape(BS, 3)
    Ksum = sum(Ks)
    outs = []
    if points is None:
        all_idx = jnp.concatenate(gidx, axis=1).reshape(-1)
        table = jnp.pad(xyz.reshape(B * N, 3), ((0, 0), (0, 13)))
        rows = _sc_gather_rows(table, all_idx)
        rows3 = rows.reshape(BS, Ksum, rows.shape[-1])
        col = 0
        for K, layers in zip(Ks, branch_params):
            gr = rows3[:, col:col + K, :].reshape(BS * K, rows.shape[-1])
            col += K
            outs.append(_sa1_branch_mlp(gr, new_flat, K, layers))
    else:
        CF = points.shape[-1]
        fused = [_fuse_mlp(layers) for layers in branch_params]
        t1, t2 = _sa2_tables(points, xyz, fused[0][0], fused[0][1],
                             fused[1][0], fused[1][1])
        all_idx = jnp.concatenate([gidx[0], gidx[1] + B * N],
                                  axis=1).reshape(-1)
        table = jnp.concatenate([t1, t2], axis=0)
        rows = _sc_gather_rows(table, all_idx)
        rows3 = rows.reshape(BS, Ksum, rows.shape[-1])
        col = 0
        for K, fl in zip(Ks, fused):
            gr = rows3[:, col:col + K, :].reshape(BS * K, rows.shape[-1])
            col += K
            a1x = fl[0][CF:CF + 3]
            outs.append(_sa2_branch_mlp(gr, new_flat, K, a1x,
                                        fl[2], fl[3], fl[4], fl[5]))
    return jnp.concatenate(outs, -1).reshape(B, S, -1)


def kernel(xyz, params):
    _, l1_xyz = _fps_pallas(xyz, 512)
    l1_points = _sa_msg_sc(xyz, None, l1_xyz, (0.1, 0.2, 0.4),
                           (32, 64, 128), params['sa1'])
    _, l2_xyz = _fps_pallas(l1_xyz, 128)
    l2_points = _sa_msg_sc(l1_xyz, l1_points, l2_xyz, (0.4, 0.8),
                           (64, 128), params['sa2'])
    l3_points = _sa3_pallas(l2_xyz, l2_points, params['sa3'])
    l2_up = _fp3_pallas(l2_points, l3_points, params['fp3'])
    l1_up = _fp2_pallas(l1_xyz, l2_xyz, l1_points, l2_up, params['fp2'])
    seg_logits, coords = _fp1_head_pallas(xyz, l1_xyz, l1_up, params['fp1'],
                                          params['head'], params['conv2'],
                                          params['coord'])
    return seg_logits, coords


# speedup vs baseline: 1.1016x; 1.0171x over previous
import functools

import jax
import jax.numpy as jnp
import numpy as np
from jax import lax
from jax.experimental import pallas as pl
from jax.experimental.pallas import tpu as pltpu
from jax.experimental.pallas import tpu_sc as plsc

_BN_EPS = 1e-5
_BIG = 1e30


def _fuse_layer(layer):
    s = layer['gamma'] / jnp.sqrt(1.0 + _BN_EPS)
    A = layer['W'].T * s[None, :]
    c = layer['b'] * s + layer['beta']
    return A, c


def _fuse_mlp(layers):
    out = []
    for l in layers:
        out.extend(_fuse_layer(l))
    return out


def _square_distance(src, dst):
    return (jnp.sum(src ** 2, -1)[:, :, None] + jnp.sum(dst ** 2, -1)[:, None, :]
            - 2.0 * jnp.einsum('bnc,bmc->bnm', src, dst))


def _index_points(points, idx):
    return jax.vmap(lambda p, i: p[i])(points, idx)


def _farthest_point_sample(xyz, npoint):
    B, N, _ = xyz.shape

    def body(i, carry):
        centroids, distance, farthest = carry
        centroids = centroids.at[:, i].set(farthest)
        centroid = jnp.take_along_axis(xyz, farthest[:, None, None], axis=1)
        dist = jnp.sum((xyz - centroid) ** 2, -1)
        distance = jnp.minimum(distance, dist)
        farthest = jnp.argmax(distance, -1).astype(jnp.int32)
        return centroids, distance, farthest

    centroids = jnp.zeros((B, npoint), jnp.int32)
    distance = jnp.full((B, N), 1e10, jnp.float32)
    farthest = jnp.zeros((B,), jnp.int32)
    centroids, _, _ = jax.lax.fori_loop(0, npoint, body,
                                        (centroids, distance, farthest))
    return centroids


def _query_ball_point(radius, nsample, xyz, new_xyz):
    B, N, _ = xyz.shape
    S = new_xyz.shape[1]
    sqrdists = _square_distance(new_xyz, xyz)
    group_idx = jnp.broadcast_to(
        jnp.arange(N, dtype=jnp.int32)[None, None, :], (B, S, N))
    group_idx = jnp.where(sqrdists > radius ** 2, N, group_idx)
    group_idx = jnp.sort(group_idx, axis=-1)[:, :, :nsample]
    group_first = jnp.broadcast_to(group_idx[:, :, :1], group_idx.shape)
    group_idx = jnp.where(group_idx == N, group_first, group_idx)
    return group_idx


def _apply_layer_xla(x, layer):
    h = jnp.einsum('...i,oi->...o', x, layer['W']) + layer['b']
    h = h * (layer['gamma'] / jnp.sqrt(1.0 + _BN_EPS)) + layer['beta']
    return jax.nn.relu(h)


def _sa_msg_xla(xyz, points, npoint, radius_list, nsample_list, branch_params,
                new_xyz=None):
    if new_xyz is None:
        fps_idx = _farthest_point_sample(xyz, npoint)
        new_xyz = _index_points(xyz, fps_idx)
    outs = []
    for radius, K, layers in zip(radius_list, nsample_list, branch_params):
        group_idx = _query_ball_point(radius, K, xyz, new_xyz)
        grouped_xyz = _index_points(xyz, group_idx) - new_xyz[:, :, None, :]
        if points is not None:
            h = jnp.concatenate([_index_points(points, group_idx), grouped_xyz], -1)
        else:
            h = grouped_xyz
        for layer in layers:
            h = _apply_layer_xla(h, layer)
        outs.append(jnp.max(h, axis=2))
    return new_xyz, jnp.concatenate(outs, -1)


def _fps_body(x_ref, y_ref, z_ref, cent_ref, nx_ref, ny_ref, nz_ref):
    B, N = x_ref.shape
    S = cent_ref.shape[1]
    x = x_ref[...]
    y = y_ref[...]
    z = z_ref[...]
    iota = jax.lax.broadcasted_iota(jnp.int32, (B, N), 1)
    iota_s = jax.lax.broadcasted_iota(jnp.int32, (B, S), 1)

    cent_ref[...] = jnp.zeros((B, S), jnp.float32)
    nx_ref[...] = jnp.zeros((B, S), jnp.float32)
    ny_ref[...] = jnp.zeros((B, S), jnp.float32)
    nz_ref[...] = jnp.zeros((B, S), jnp.float32)

    def body(i, carry):
        distance, farthest = carry
        slotf = (iota_s == i).astype(jnp.float32)
        cent_ref[...] += slotf * farthest.astype(jnp.float32)
        onehot = (iota == farthest)
        cx = jnp.sum(jnp.where(onehot, x, 0.0), axis=1, keepdims=True)
        cy = jnp.sum(jnp.where(onehot, y, 0.0), axis=1, keepdims=True)
        cz = jnp.sum(jnp.where(onehot, z, 0.0), axis=1, keepdims=True)
        nx_ref[...] += slotf * cx
        ny_ref[...] += slotf * cy
        nz_ref[...] += slotf * cz
        dx = x - cx
        dy = y - cy
        dz = z - cz
        dist = (dx * dx + dy * dy) + dz * dz
        distance = jnp.minimum(distance, dist)
        m = jnp.max(distance, axis=1, keepdims=True)
        farthest = jnp.min(jnp.where(distance == m, iota, N),
                           axis=1, keepdims=True)
        return distance, farthest

    jax.lax.fori_loop(
        0, S, body,
        (jnp.full((B, N), 1e10, jnp.float32), jnp.zeros((B, 1), jnp.int32)))


def _fps_pallas(xyz, npoint):
    B, N, _ = xyz.shape
    x = xyz[:, :, 0]
    y = xyz[:, :, 1]
    z = xyz[:, :, 2]
    sds = jax.ShapeDtypeStruct
    cent, nx, ny, nz = pl.pallas_call(
        _fps_body,
        out_shape=(sds((B, npoint), jnp.float32), sds((B, npoint), jnp.float32),
                   sds((B, npoint), jnp.float32), sds((B, npoint), jnp.float32)),
    )(x, y, z)
    return cent.astype(jnp.int32), jnp.stack([nx, ny, nz], axis=-1)


def _sa3_body(xyz_ref, pts_ref, a1, c1, a2, c2, a3, c3, out_ref):
    B, S, _ = xyz_ref.shape
    h = jnp.concatenate([xyz_ref[...], pts_ref[...]], axis=-1)
    h = h.reshape(B * S, h.shape[-1])
    h = jnp.maximum(jnp.dot(h, a1[...], preferred_element_type=jnp.float32)
                    + c1[...][0], 0.0)
    h = jnp.maximum(jnp.dot(h, a2[...], preferred_element_type=jnp.float32)
                    + c2[...][0], 0.0)
    h = jnp.maximum(jnp.dot(h, a3[...], preferred_element_type=jnp.float32)
                    + c3[...][0], 0.0)
    h = h.reshape(B, S, h.shape[-1])
    out_ref[...] = jnp.max(h, axis=1)


def _sa3_pallas(l2_xyz, l2_points, layers):
    B, S, _ = l2_xyz.shape
    a1, c1, a2, c2, a3, c3 = _fuse_mlp(layers)
    cout = a3.shape[1]
    return pl.pallas_call(
        _sa3_body,
        out_shape=jax.ShapeDtypeStruct((B, cout), jnp.float32),
    )(l2_xyz, l2_points, a1, c1[None], a2, c2[None], a3, c3[None])


def _fp3_body(pts_ref, glob_ref, a1, c1, a2, c2, out_ref):
    B, S, C = pts_ref.shape
    G = glob_ref.shape[-1]
    glob = jnp.broadcast_to(glob_ref[...][:, None, :], (B, S, G))
    h = jnp.concatenate([pts_ref[...], glob], axis=-1).reshape(B * S, C + G)
    h = jnp.maximum(jnp.dot(h, a1[...], preferred_element_type=jnp.float32)
                    + c1[...][0], 0.0)
    h = jnp.maximum(jnp.dot(h, a2[...], preferred_element_type=jnp.float32)
                    + c2[...][0], 0.0)
    out_ref[...] = h.reshape(B, S, h.shape[-1])


def _fp3_pallas(l2_points, l3_points, layers):
    B, S, _ = l2_points.shape
    a1, c1, a2, c2 = _fuse_mlp(layers)
    cout = a2.shape[1]
    return pl.pallas_call(
        _fp3_body,
        out_shape=jax.ShapeDtypeStruct((B, S, cout), jnp.float32),
    )(l2_points, l3_points, a1, c1[None], a2, c2[None])


def _top3_interp(d, pts2):
    R, S = d.shape
    iota = jax.lax.broadcasted_iota(jnp.int32, (R, S), 1)
    wmat = jnp.zeros((R, S), jnp.float32)
    recips = []
    onehots = []
    dd = d
    for _ in range(3):
        m = jnp.min(dd, axis=-1, keepdims=True)
        sel = (dd == m)
        first = jnp.min(jnp.where(sel, iota, S), axis=-1, keepdims=True)
        hit = (iota == first)
        recips.append(1.0 / (jnp.maximum(m, 0.0) + 1e-8))
        onehots.append(hit)
        dd = jnp.where(hit, _BIG, dd)
    norm = recips[0] + recips[1] + recips[2]
    for r, hit in zip(recips, onehots):
        wmat = wmat + jnp.where(hit, r / norm, 0.0)
    return jnp.dot(wmat, pts2, preferred_element_type=jnp.float32)


def _fp2_body(x1_ref, x2_ref, p1_ref, p2_ref, a1, c1, a2, c2, out_ref):
    x1 = x1_ref[0]
    x2 = x2_ref[0]
    d = (jnp.sum(x1 * x1, -1, keepdims=True)
         + jnp.sum(x2 * x2, -1)[None, :]
         - 2.0 * jax.lax.dot_general(x1, x2, (((1,), (1,)), ((), ())),
                                     preferred_element_type=jnp.float32))
    interp = _top3_interp(d, p2_ref[0])
    h = jnp.concatenate([p1_ref[0], interp], axis=-1)
    h = jnp.maximum(jnp.dot(h, a1[...], preferred_element_type=jnp.float32)
                    + c1[...][0], 0.0)
    h = jnp.maximum(jnp.dot(h, a2[...], preferred_element_type=jnp.float32)
                    + c2[...][0], 0.0)
    out_ref[0] = h


def _fp2_pallas(xyz1, xyz2, points1, points2, layers):
    B, N, _ = xyz1.shape
    S = xyz2.shape[1]
    C1 = points1.shape[-1]
    C2 = points2.shape[-1]
    a1, c1, a2, c2 = _fuse_mlp(layers)
    cout = a2.shape[1]
    return pl.pallas_call(
        _fp2_body,
        grid=(B,),
        in_specs=[
            pl.BlockSpec((1, N, 3), lambda b: (b, 0, 0)),
            pl.BlockSpec((1, S, 3), lambda b: (b, 0, 0)),
            pl.BlockSpec((1, N, C1), lambda b: (b, 0, 0)),
            pl.BlockSpec((1, S, C2), lambda b: (b, 0, 0)),
            pl.BlockSpec(a1.shape, lambda b: (0, 0)),
            pl.BlockSpec((1,) + c1.shape, lambda b: (0, 0)),
            pl.BlockSpec(a2.shape, lambda b: (0, 0)),
            pl.BlockSpec((1,) + c2.shape, lambda b: (0, 0)),
        ],
        out_specs=pl.BlockSpec((1, N, cout), lambda b: (b, 0, 0)),
        out_shape=jax.ShapeDtypeStruct((B, N, cout), jnp.float32),
    )(xyz1, xyz2, points1, points2, a1, c1[None], a2, c2[None])


def _fp1_head_body(x1_ref, x2_ref, p2_ref, a1, c1, a2, c2, ah, ch, af, cf,
                   out_ref):
    x1 = x1_ref[0]
    x2 = x2_ref[0]
    d = (jnp.sum(x1 * x1, -1, keepdims=True)
         + jnp.sum(x2 * x2, -1)[None, :]
         - 2.0 * jax.lax.dot_general(x1, x2, (((1,), (1,)), ((), ())),
                                     preferred_element_type=jnp.float32))
    interp = _top3_interp(d, p2_ref[0])
    h = jnp.concatenate([x1, interp], axis=-1)
    h = jnp.maximum(jnp.dot(h, a1[...], preferred_element_type=jnp.float32)
                    + c1[...][0], 0.0)
    h = jnp.maximum(jnp.dot(h, a2[...], preferred_element_type=jnp.float32)
                    + c2[...][0], 0.0)
    h = jnp.maximum(jnp.dot(h, ah[...], preferred_element_type=jnp.float32)
                    + ch[...][0], 0.0)
    out_ref[0] = jnp.dot(h, af[...], preferred_element_type=jnp.float32) \
        + cf[...][0]


def _fp1_head_pallas(xyz, l1_xyz, l1_up, fp1_layers, head_layers, conv2, coord):
    B, N, _ = xyz.shape
    S = l1_xyz.shape[1]
    C2 = l1_up.shape[-1]
    a1, c1, a2, c2 = _fuse_mlp(fp1_layers)
    ah, ch = _fuse_layer(head_layers[0])
    wf = jnp.concatenate([conv2['W'], coord['W']], axis=0)
    bf = jnp.concatenate([conv2['b'], coord['b']], axis=0)
    af = jnp.zeros((wf.shape[1], 128), jnp.float32).at[:, :6].set(wf.T)
    cf = jnp.zeros((128,), jnp.float32).at[:6].set(bf)
    RC = 1024
    out = pl.pallas_call(
        _fp1_head_body,
        grid=(B, N // RC),
        in_specs=[
            pl.BlockSpec((1, RC, 3), lambda b, r: (b, r, 0)),
            pl.BlockSpec((1, S, 3), lambda b, r: (b, 0, 0)),
            pl.BlockSpec((1, S, C2), lambda b, r: (b, 0, 0)),
            pl.BlockSpec(a1.shape, lambda b, r: (0, 0)),
            pl.BlockSpec((1,) + c1.shape, lambda b, r: (0, 0)),
            pl.BlockSpec(a2.shape, lambda b, r: (0, 0)),
            pl.BlockSpec((1,) + c2.shape, lambda b, r: (0, 0)),
            pl.BlockSpec(ah.shape, lambda b, r: (0, 0)),
            pl.BlockSpec((1,) + ch.shape, lambda b, r: (0, 0)),
            pl.BlockSpec(af.shape, lambda b, r: (0, 0)),
            pl.BlockSpec((1,) + cf.shape, lambda b, r: (0, 0)),
        ],
        out_specs=pl.BlockSpec((1, RC, 128), lambda b, r: (b, r, 0)),
        out_shape=jax.ShapeDtypeStruct((B, N, 128), jnp.float32),
    )(xyz, l1_xyz, l1_up, a1, c1[None], a2, c2[None], ah, ch[None], af,
      cf[None])
    return out[..., :3], out[..., 3:6]


def _mask_body(new_ref, xyz_ref, out_ref, *, radii):
    new = new_ref[0]
    xyz = xyz_ref[0]
    N = xyz.shape[0]
    N8 = N // 8
    nb = len(radii)
    d = (jnp.sum(new * new, -1, keepdims=True)
         + jnp.sum(xyz * xyz, -1)[None, :]
         - 2.0 * lax.dot_general(new, xyz, (((1,), (1,)), ((), ())),
                                 preferred_element_type=jnp.float32))
    b = jnp.zeros(d.shape, jnp.int32)
    for bi, r in enumerate(radii):
        b = b + jnp.where(d > r * r, 0, 1 << bi)
    w = b[:, :N8]
    for u in range(1, 8):
        w = w | (b[:, u * N8:(u + 1) * N8] << (nb * u))
    out_ref[0] = w


def _radius_masks(new_xyz, xyz, radii):
    B, S, _ = new_xyz.shape
    N = xyz.shape[1]
    R = min(S, 128)
    return pl.pallas_call(
        functools.partial(_mask_body, radii=radii),
        grid=(B, S // R),
        in_specs=[
            pl.BlockSpec((1, R, 3), lambda b, s: (b, s, 0)),
            pl.BlockSpec((1, N, 3), lambda b, s: (b, 0, 0)),
        ],
        out_specs=pl.BlockSpec((1, R, N // 8), lambda b, s: (b, s, 0)),
        out_shape=jax.ShapeDtypeStruct((B, S, N // 8), jnp.int32),
    )(new_xyz, xyz)


def _sc_info():
    info = plsc.get_sparse_core_info()
    return info.num_cores, info.num_subcores


def _sc_ball_compact(mw_flat, B, S, N, Ks):
    NC, NS = _sc_info()
    NW = NC * NS
    G = (B * S) // 16
    GPW = G // NW
    GPS = S // 16
    N8 = N // 8
    nb = len(Ks)
    mesh = plsc.VectorSubcoreMesh(core_axis_name="c", subcore_axis_name="s")
    sds = jax.ShapeDtypeStruct
    scratch = [pltpu.VMEM((N8 * 16,), jnp.int32)]
    scratch += [pltpu.VMEM((16, K), jnp.int32) for K in Ks]

    @functools.partial(
        pl.kernel, mesh=mesh,
        out_type=tuple(sds((B * S, K), jnp.int32) for K in Ks),
        scratch_types=scratch,
        compiler_params=pltpu.CompilerParams(needs_layout_passes=False))
    def k(mw_hbm, *refs):
        outs = refs[:len(Ks)]
        mwv = refs[len(Ks)]
        bufs = refs[len(Ks) + 1:]
        wid = lax.axis_index("s") * NC + lax.axis_index("c")
        iota16 = lax.iota(jnp.int32, 16)
        zero16 = jnp.zeros((16,), jnp.int32)
        rowbase = iota16 * N8

        def group_body(gi, _):
            g = wid * GPW + gi
            b = g // GPS
            pltpu.sync_copy(mw_hbm.at[pl.ds(g * 16 * N8, 16 * N8)], mwv)
            base_val = b * N

            def ubody(u, offs):
                base_u = base_val + u * N8
                shift = nb * u

                def jbody(jw, offs):
                    mv = plsc.load_gather(mwv, [rowbase + jw]) >> shift
                    val = zero16 + (base_u + jw)
                    new_offs = []
                    for bi, K in enumerate(Ks):
                        m = (mv & (1 << bi)) != 0
                        can = m & (offs[bi] < K)
                        col = jnp.minimum(offs[bi], K - 1)
                        plsc.store_scatter(bufs[bi], [iota16, col], val,
                                           mask=can)
                        new_offs.append(offs[bi] + jnp.where(can, 1, 0))
                    return tuple(new_offs)

                return lax.fori_loop(0, N8, jbody, offs)

            offs = lax.fori_loop(0, 8, ubody,
                                 tuple(zero16 for _ in Ks))
            for bi, K in enumerate(Ks):
                first0 = plsc.load_gather(bufs[bi], [iota16, zero16])

                def fbody(kk, _, bi=bi, K=K, first0=first0):
                    need = offs[bi] <= kk
                    plsc.store_scatter(bufs[bi], [iota16, zero16 + kk],
                                       first0, mask=need)
                    return 0

                lax.fori_loop(1, K, fbody, 0)
                pltpu.sync_copy(bufs[bi], outs[bi].at[pl.ds(g * 16, 16)])
            return 0

        lax.fori_loop(0, GPW, group_body, 0)

    return k(mw_flat)


def _sc_gather_rows(table, idx):
    NC, NS = _sc_info()
    NW = NC * NS
    M = idx.shape[0]
    D = table.shape[1]
    m_per_w = M // NW
    ch = m_per_w
    while ch * D * 4 > 260000 or ch > 4096:
        ch //= 2
    nch = m_per_w // ch
    mesh = plsc.VectorSubcoreMesh(core_axis_name="c", subcore_axis_name="s")

    @functools.partial(
        pl.kernel, mesh=mesh,
        out_type=jax.ShapeDtypeStruct((M, D), jnp.float32),
        scratch_types=[pltpu.VMEM((ch,), jnp.int32),
                       pltpu.VMEM((ch, D), jnp.float32),
                       pltpu.SemaphoreType.DMA],
        compiler_params=pltpu.CompilerParams(needs_layout_passes=False,
                                             use_tc_tiling_on_sc=False))
    def k(table_hbm, idx_hbm, out_hbm, idx_v, rows_v, sem):
        wid = lax.axis_index("s") * NC + lax.axis_index("c")
        base = wid * m_per_w

        def body(c, _):
            off = base + c * ch
            pltpu.sync_copy(idx_hbm.at[pl.ds(off, ch)], idx_v)
            pltpu.async_copy(table_hbm.at[idx_v], rows_v, sem).wait()
            pltpu.sync_copy(rows_v, out_hbm.at[pl.ds(off, ch)])
            return 0

        lax.fori_loop(0, nch, body, 0)

    return k(table, idx)


def _sa1_mlp_body(g_ref, new_ref, a1, c1, a2, c2, a3, c3, out_ref, *, K):
    RK = g_ref.shape[0]
    R = RK // K
    new = new_ref[...]
    t = g_ref[...].reshape(R, K, 16)[:, :, :3] - new[:, None, :]
    h = t.reshape(RK, 3)
    h = jnp.maximum(jnp.dot(h, a1[...], preferred_element_type=jnp.float32)
                    + c1[...][0], 0.0)
    h = jnp.maximum(jnp.dot(h, a2[...], preferred_element_type=jnp.float32)
                    + c2[...][0], 0.0)
    h = jnp.maximum(jnp.dot(h, a3[...], preferred_element_type=jnp.float32)
                    + c3[...][0], 0.0)
    out_ref[...] = jnp.max(h.reshape(R, K, h.shape[-1]), axis=1)


def _sa1_branch_mlp(gathered, new_flat, K, layers):
    BS = new_flat.shape[0]
    a1, c1, a2, c2, a3, c3 = _fuse_mlp(layers)
    cout = a3.shape[1]
    R = 64
    return pl.pallas_call(
        functools.partial(_sa1_mlp_body, K=K),
        grid=(BS // R,),
        in_specs=[
            pl.BlockSpec((R * K, 16), lambda i: (i, 0)),
            pl.BlockSpec((R, 3), lambda i: (i, 0)),
            pl.BlockSpec(a1.shape, lambda i: (0, 0)),
            pl.BlockSpec((1,) + c1.shape, lambda i: (0, 0)),
            pl.BlockSpec(a2.shape, lambda i: (0, 0)),
            pl.BlockSpec((1,) + c2.shape, lambda i: (0, 0)),
            pl.BlockSpec(a3.shape, lambda i: (0, 0)),
            pl.BlockSpec((1,) + c3.shape, lambda i: (0, 0)),
        ],
        out_specs=pl.BlockSpec((R, cout), lambda i: (i, 0)),
        out_shape=jax.ShapeDtypeStruct((BS, cout), jnp.float32),
    )(gathered, new_flat, a1, c1[None], a2, c2[None], a3, c3[None])


def _sa2_tables_body(pts_ref, xyz_ref, a1_b1, c1_b1, a1_b2, c1_b2, t1_ref,
                     t2_ref):
    BN = t1_ref.shape[0]
    h = jnp.concatenate([pts_ref[...], xyz_ref[...]], axis=-1)
    h = h.reshape(BN, h.shape[-1])
    t1_ref[...] = jnp.dot(h, a1_b1[...], preferred_element_type=jnp.float32) \
        + c1_b1[...][0]
    t2_ref[...] = jnp.dot(h, a1_b2[...], preferred_element_type=jnp.float32) \
        + c1_b2[...][0]


def _sa2_tables(points, xyz, a1_b1, c1_b1, a1_b2, c1_b2):
    B, N, CF = points.shape
    cout = a1_b1.shape[1]
    sds = jax.ShapeDtypeStruct
    return pl.pallas_call(
        _sa2_tables_body,
        out_shape=(sds((B * N, cout), jnp.float32),
                   sds((B * N, cout), jnp.float32)),
    )(points, xyz, a1_b1, c1_b1[None], a1_b2, c1_b2[None])


def _sa2_mlp_body(g_ref, new_ref, a1x, a2, c2, a3, c3, out_ref, *, K):
    RK = g_ref.shape[0]
    R = RK // K
    ca = jnp.dot(new_ref[...], a1x[...], preferred_element_type=jnp.float32)
    h = g_ref[...].reshape(R, K, g_ref.shape[-1]) - ca[:, None, :]
    h = jnp.maximum(h.reshape(RK, h.shape[-1]), 0.0)
    h = jnp.maximum(jnp.dot(h, a2[...], preferred_element_type=jnp.float32)
                    + c2[...][0], 0.0)
    h = jnp.maximum(jnp.dot(h, a3[...], preferred_element_type=jnp.float32)
                    + c3[...][0], 0.0)
    out_ref[...] = jnp.max(h.reshape(R, K, h.shape[-1]), axis=1)


def _sa2_branch_mlp(gathered, new_flat, K, a1x, a2, c2, a3, c3):
    BS = new_flat.shape[0]
    C1 = gathered.shape[1]
    cout = a3.shape[1]
    R = 32
    return pl.pallas_call(
        functools.partial(_sa2_mlp_body, K=K),
        grid=(BS // R,),
        in_specs=[
            pl.BlockSpec((R * K, C1), lambda i: (i, 0)),
            pl.BlockSpec((R, 3), lambda i: (i, 0)),
            pl.BlockSpec(a1x.shape, lambda i: (0, 0)),
            pl.BlockSpec(a2.shape, lambda i: (0, 0)),
            pl.BlockSpec((1,) + c2.shape, lambda i: (0, 0)),
            pl.BlockSpec(a3.shape, lambda i: (0, 0)),
            pl.BlockSpec((1,) + c3.shape, lambda i: (0, 0)),
        ],
        out_specs=pl.BlockSpec((R, cout), lambda i: (i, 0)),
        out_shape=jax.ShapeDtypeStruct((BS, cout), jnp.float32),
    )(gathered, new_flat, a1x, a2, c2[None], a3, c3[None])


def _sa_msg_sc(xyz, points, new_xyz, radii, Ks, branch_params):
    B, N, _ = xyz.shape
    S = new_xyz.shape[1]
    BS = B * S
    mw = _radius_masks(new_xyz, xyz, radii)
    gidx = _sc_ball_compact(mw.reshape(B * S * (N // 8)), B, S, N, Ks)
    new_flat = new_xyz.reshape(BS, 3)
    Ksum = sum(Ks)
    outs = []
    if points is None:
        all_idx = jnp.concatenate(gidx, axis=1).reshape(-1)
        table = jnp.pad(xyz.reshape(B * N, 3), ((0, 0), (0, 13)))
        rows = _sc_gather_rows(table, all_idx)
        rows3 = rows.reshape(BS, Ksum, rows.shape[-1])
        col = 0
        for K, layers in zip(Ks, branch_params):
            gr = rows3[:, col:col + K, :].reshape(BS * K, rows.shape[-1])
            col += K
            outs.append(_sa1_branch_mlp(gr, new_flat, K, layers))
    else:
        CF = points.shape[-1]
        fused = [_fuse_mlp(layers) for layers in branch_params]
        t1, t2 = _sa2_tables(points, xyz, fused[0][0], fused[0][1],
                             fused[1][0], fused[1][1])
        all_idx = jnp.concatenate([gidx[0], gidx[1] + B * N],
                                  axis=1).reshape(-1)
        table = jnp.concatenate([t1, t2], axis=0)
        rows = _sc_gather_rows(table, all_idx)
        rows3 = rows.reshape(BS, Ksum, rows.shape[-1])
        col = 0
        for K, fl in zip(Ks, fused):
            gr = rows3[:, col:col + K, :].reshape(BS * K, rows.shape[-1])
            col += K
            a1x = fl[0][CF:CF + 3]
            outs.append(_sa2_branch_mlp(gr, new_flat, K, a1x,
                                        fl[2], fl[3], fl[4], fl[5]))
    return jnp.concatenate(outs, -1).reshape(B, S, -1)


def kernel(xyz, params):
    _, l1_xyz = _fps_pallas(xyz, 512)
    l1_points = _sa_msg_sc(xyz, None, l1_xyz, (0.1, 0.2, 0.4),
                           (32, 64, 128), params['sa1'])
    _, l2_xyz = _fps_pallas(l1_xyz, 128)
    l2_points = _sa_msg_sc(l1_xyz, l1_points, l2_xyz, (0.4, 0.8),
                           (64, 128), params['sa2'])
    l3_points = _sa3_pallas(l2_xyz, l2_points, params['sa3'])
    l2_up = _fp3_pallas(l2_points, l3_points, params['fp3'])
    l1_up = _fp2_pallas(l1_xyz, l2_xyz, l1_points, l2_up, params['fp2'])
    seg_logits, coords = _fp1_head_pallas(xyz, l1_xyz, l1_up, params['fp1'],
                                          params['head'], params['conv2'],
                                          params['coord'])
    return seg_logits, coords
